# Initial kernel scaffold; baseline (speedup 1.0000x reference)
import jax
import jax.numpy as jnp
from jax.experimental import pallas as pl


def kernel(x, table):
    raise NotImplementedError("write your pallas kernel here")


# speedup vs baseline: 1.1852x; 1.1852x over previous
import functools

import jax
import jax.numpy as jnp
from jax import lax
from jax.experimental import pallas as pl
from jax.experimental.pallas import tpu as pltpu
from jax.experimental.pallas import tpu_sc as plsc

VOCAB = 1000000
EMBD = 32
B = 4096
L = 200

NC = 2
NS = 16
NW = NC * NS

TOTAL = B * L
PER_W = TOTAL // NW
R = 128
NBLK = PER_W // R


def _make_kernel():
    mesh = plsc.VectorSubcoreMesh(core_axis_name="c", subcore_axis_name="s")

    @functools.partial(
        pl.kernel,
        out_type=jax.ShapeDtypeStruct((TOTAL, EMBD), jnp.float32),
        mesh=mesh,
        compiler_params=pltpu.CompilerParams(use_tc_tiling_on_sc=False),
        scratch_types=[
            pltpu.VMEM((NBLK, R), jnp.int32),
            pltpu.VMEM((R, EMBD), jnp.float32),
            pltpu.SemaphoreType.DMA,
        ],
    )
    def emb_kernel(table_hbm, x_hbm, out_hbm, idx_v, rows_v, gsem):
        wid = lax.axis_index("s") * NC + lax.axis_index("c")
        base = wid * PER_W
        pltpu.sync_copy(x_hbm.at[wid], idx_v)

        @pl.loop(0, NBLK)
        def _blocks(j):
            pltpu.async_copy(table_hbm.at[idx_v.at[j]], rows_v, gsem).wait()

            @pl.loop(0, R)
            def _relu(i):
                rows_v[i, 0:16] = jnp.maximum(rows_v[i, 0:16], 0.0)
                rows_v[i, 16:32] = jnp.maximum(rows_v[i, 16:32], 0.0)

            pltpu.sync_copy(rows_v, out_hbm.at[pl.ds(base + j * R, R)])

    return emb_kernel


_EMB_KERNEL = _make_kernel()


@jax.jit
def kernel(x, table):
    x_flat = x.astype(jnp.int32).reshape(NW, NBLK, R)
    out = _EMB_KERNEL(table, x_flat)
    return out.reshape(B, L, EMBD)


# speedup vs baseline: 1.4764x; 1.2458x over previous
import functools

import jax
import jax.numpy as jnp
from jax import lax
from jax.experimental import pallas as pl
from jax.experimental.pallas import tpu as pltpu
from jax.experimental.pallas import tpu_sc as plsc

VOCAB = 1000000
EMBD = 32
B = 4096
L = 200

NC = 2
NS = 16
NW = NC * NS

TOTAL = B * L
PER_W = TOTAL // NW
R = 128
NBLK = PER_W // R
K = 10
GROUP = K * R
NGRP = NBLK // K


def _make_kernel():
    mesh = plsc.VectorSubcoreMesh(core_axis_name="c", subcore_axis_name="s")

    @functools.partial(
        pl.kernel,
        out_type=jax.ShapeDtypeStruct((TOTAL, EMBD), jnp.float32),
        mesh=mesh,
        compiler_params=pltpu.CompilerParams(use_tc_tiling_on_sc=False),
        scratch_types=[
            pltpu.VMEM((NBLK, R), jnp.int32),
            pltpu.VMEM((GROUP, EMBD), jnp.float32),
            pltpu.VMEM((GROUP, EMBD), jnp.float32),
            pltpu.SemaphoreType.DMA,
            pltpu.SemaphoreType.DMA,
            pltpu.SemaphoreType.DMA,
            pltpu.SemaphoreType.DMA,
        ],
    )
    def emb_kernel(table_hbm, x_hbm, out_hbm, idx_v, buf0, buf1, g0, g1, s0, s1):
        bufs = (buf0, buf1)
        gsem = (g0, g1)
        ssem = (s0, s1)
        wid = lax.axis_index("s") * NC + lax.axis_index("c")
        base = wid * PER_W
        pltpu.sync_copy(x_hbm.at[wid], idx_v)

        def gather_start(g, s):
            for i in range(K):
                pltpu.async_copy(
                    table_hbm.at[idx_v.at[g * K + i]],
                    bufs[s].at[pl.ds(i * R, R)],
                    gsem[s],
                )

        def gather_wait(g, s):
            for i in range(K):
                pltpu.make_async_copy(
                    table_hbm.at[idx_v.at[g * K + i]],
                    bufs[s].at[pl.ds(i * R, R)],
                    gsem[s],
                ).wait()

        def store_start(g, s):
            pltpu.async_copy(
                bufs[s], out_hbm.at[pl.ds(base + g * GROUP, GROUP)], ssem[s]
            )

        def store_wait(g, s):
            pltpu.make_async_copy(
                bufs[s], out_hbm.at[pl.ds(base + g * GROUP, GROUP)], ssem[s]
            ).wait()

        def relu(s):
            buf = bufs[s]

            @pl.loop(0, GROUP, unroll=4)
            def _rows(i):
                buf[i, 0:16] = jnp.maximum(buf[i, 0:16], 0.0)
                buf[i, 16:32] = jnp.maximum(buf[i, 16:32], 0.0)

        gather_start(0, 0)

        @pl.loop(0, NGRP, step=2)
        def _pair(G):
            for s in range(2):
                g = G + s
                o = 1 - s

                @pl.when(g >= 1)
                def _drain_prev_store():
                    store_wait(g - 1, o)

                @pl.when(g + 1 < NGRP)
                def _fire_next_gather():
                    gather_start(g + 1, o)

                gather_wait(g, s)
                relu(s)
                store_start(g, s)

        store_wait(NGRP - 1, 1)

    return emb_kernel


_EMB_KERNEL = _make_kernel()


@jax.jit
def kernel(x, table):
    x_flat = x.astype(jnp.int32).reshape(NW, NBLK, R)
    out = _EMB_KERNEL(table, x_flat)
    return out.reshape(B, L, EMBD)
